# Initial kernel scaffold; baseline (speedup 1.0000x reference)
import jax
import jax.numpy as jnp
from jax.experimental import pallas as pl


def kernel(anchor, feature):
    raise NotImplementedError("write your pallas kernel here")


# speedup vs baseline: 301.5102x; 301.5102x over previous
import jax
import jax.numpy as jnp
from jax.experimental import pallas as pl

_ATOMS = 51
_V_MIN = -10.0
_V_MAX = 10.0


def _projection_matrix():
    atoms = _ATOMS
    delta = (_V_MAX - _V_MIN) / (atoms - 1)
    supports = jnp.linspace(_V_MIN, _V_MAX, atoms).astype(jnp.float32)
    tz = jnp.clip(supports, _V_MIN, _V_MAX)
    b = (tz - _V_MIN) / delta
    l = jnp.floor(b).astype(jnp.int32)
    u = jnp.ceil(b).astype(jnp.int32)
    l = jnp.where((u > 0) & (l == u), l - 1, l)
    u = jnp.where((l < atoms - 1) & (l == u), u + 1, u)
    wl = u.astype(jnp.float32) - b
    wu = b - l.astype(jnp.float32)
    cols = jnp.arange(atoms)
    mc = jnp.zeros((atoms, atoms), jnp.float32)
    mc = mc.at[l, cols].add(wl)
    mc = mc.at[u, cols].add(wu)
    return mc


def _loss_block_kernel(a_ref, f_ref, m_ref, o_ref):
    i = pl.program_id(0)
    g = jnp.log(f_ref[...] + 1e-16)
    gl = jnp.dot(g, m_ref[...], preferred_element_type=jnp.float32)
    part = jnp.sum(a_ref[...] * gl, keepdims=True)

    @pl.when(i == 0)
    def _init():
        o_ref[...] = jnp.zeros_like(o_ref)

    o_ref[...] += part


def kernel(anchor, feature):
    batch, atoms = anchor.shape
    mc = _projection_matrix()
    blk = 4096
    grid = batch // blk
    total = pl.pallas_call(
        _loss_block_kernel,
        grid=(grid,),
        in_specs=[
            pl.BlockSpec((blk, atoms), lambda i: (i, 0)),
            pl.BlockSpec((blk, atoms), lambda i: (i, 0)),
            pl.BlockSpec((atoms, atoms), lambda i: (0, 0)),
        ],
        out_specs=pl.BlockSpec((1, 1), lambda i: (0, 0)),
        out_shape=jax.ShapeDtypeStruct((1, 1), jnp.float32),
    )(anchor, feature, mc)
    return -(total[0, 0] / batch)
